# baseline (speedup 1.0000x reference)
import functools

import jax
import jax.numpy as jnp
from jax import lax
from jax.experimental import pallas as pl
from jax.experimental.pallas import tpu as pltpu
from jax.experimental.pallas import tpu_sc as plsc

VOCAB = 17
EMB_DIM = 32
CONTEXT = 2
IN_DIM = EMB_DIM * CONTEXT

_mesh = plsc.VectorSubcoreMesh(core_axis_name="c", subcore_axis_name="s")


def _full(v):
    return jnp.full((16,), v, jnp.int32)


@functools.partial(
    pl.kernel,
    out_type=jax.ShapeDtypeStruct((1, VOCAB), jnp.float32),
    mesh=_mesh,
    scratch_types=[
        pltpu.VMEM((CONTEXT,), jnp.int32),
        pltpu.VMEM((CONTEXT, EMB_DIM), jnp.float32),
        pltpu.VMEM((VOCAB, IN_DIM), jnp.float32),
        pltpu.VMEM((32,), jnp.float32),
        pltpu.VMEM((VOCAB,), jnp.float32),
        pltpu.SemaphoreType.DMA,
        pltpu.SemaphoreType.DMA,
    ],
    compiler_params=pltpu.CompilerParams(
        needs_layout_passes=False, use_tc_tiling_on_sc=False),
)
def _sc_kernel(x_hbm, emb_hbm, w_hbm, b_hbm, out_hbm,
               x_v, rows_v, w_v, b_v, out_v, sem0, sem1):
    wid = lax.axis_index("s") * 2 + lax.axis_index("c")

    @pl.when(wid == 0)
    def _():
        cpx = pltpu.async_copy(x_hbm, x_v, sem0)
        cpw = pltpu.async_copy(w_hbm, w_v, sem1)
        cpb = pltpu.async_copy(b_hbm, b_v.at[pl.ds(0, VOCAB)], sem1)
        cpx.wait()
        cpe = pltpu.async_copy(emb_hbm.at[x_v], rows_v, sem0)
        cpe.wait()
        cpw.wait()
        cpb.wait()

        lane = jnp.arange(16, dtype=jnp.int32)
        e_regs = [rows_v[c, pl.ds(p * 16, 16)]
                  for c in range(CONTEXT) for p in range(EMB_DIM // 16)]
        y_lo = b_v[pl.ds(0, 16)]
        for kk in range(IN_DIM):
            ek = jnp.take_along_axis(e_regs[kk // 16], _full(kk % 16), axis=0)
            col = plsc.load_gather(w_v, [lane, _full(kk)])
            y_lo = y_lo + ek * col
        out_v[pl.ds(0, 16)] = y_lo

        s = jnp.zeros((16,), jnp.float32)
        for c in range(CONTEXT):
            for p in range(EMB_DIM // 16):
                e_part = rows_v[c, pl.ds(p * 16, 16)]
                w_part = w_v[VOCAB - 1, pl.ds(c * EMB_DIM + p * 16, 16)]
                s = s + e_part * w_part
        y16v = b_v[pl.ds(16, 16)] + jnp.sum(s)
        plsc.store_scatter(out_v, [_full(VOCAB - 1)], y16v, mask=lane == 0)

        pltpu.sync_copy(out_v, out_hbm.at[0])


def kernel(x, emb, W, b):
    return _sc_kernel(x.astype(jnp.int32), emb, W, b)


# speedup vs baseline: 13.6470x; 13.6470x over previous
import jax
import jax.numpy as jnp
from jax.experimental import pallas as pl
from jax.experimental.pallas import tpu as pltpu

VOCAB = 17
EMB_DIM = 32
CONTEXT = 2
IN_DIM = EMB_DIM * CONTEXT


def _body(x_ref, emb_ref, w_ref, b_ref, out_ref):
    e0 = emb_ref[pl.ds(x_ref[0], 1), :]
    e1 = emb_ref[pl.ds(x_ref[1], 1), :]
    e = jnp.concatenate([e0, e1], axis=1)
    y = jax.lax.dot_general(
        e, w_ref[...],
        dimension_numbers=(((1,), (1,)), ((), ())),
        preferred_element_type=jnp.float32,
    )
    out_ref[...] = y + b_ref[...]


def kernel(x, emb, W, b):
    return pl.pallas_call(
        _body,
        out_shape=jax.ShapeDtypeStruct((1, VOCAB), jnp.float32),
        in_specs=[
            pl.BlockSpec(memory_space=pltpu.SMEM),
            pl.BlockSpec(memory_space=pltpu.VMEM),
            pl.BlockSpec(memory_space=pltpu.VMEM),
            pl.BlockSpec(memory_space=pltpu.VMEM),
        ],
        out_specs=pl.BlockSpec(memory_space=pltpu.VMEM),
    )(x.astype(jnp.int32), emb, W, b.reshape(1, VOCAB))
